# baseline (speedup 1.0000x reference)
import math

import numpy as np
import jax
import jax.numpy as jnp
from jax.experimental import pallas as pl
from jax.experimental.pallas import tpu as pltpu


_VMEM_LIMIT = 48 * 1024 * 1024


def _tap_offsets(dp, p):
    return tuple(p + (kd - 1) * dp * dp + (kh - 1) * dp + (kw - 1)
                 for kd in range(3) for kh in range(3) for kw in range(3))


def _halo_mask_t(dp):
    m = np.zeros((dp, dp, dp), np.float32)
    m[1:-1, 1:-1, 1:-1] = 1.0
    return jnp.asarray(m.reshape(1, dp * dp * dp))


def _pick_chunk(sp, cap=2048):
    if sp <= cap:
        return sp
    for n in range(2, sp + 1):
        if sp % n == 0 and sp // n <= cap:
            return sp // n
    return sp


def _dot_ta(w, im):
    return jax.lax.dot_general(w, im, (((0,), (0,)), ((), ())),
                               preferred_element_type=jnp.float32)


def _tconv(srcs, w_ref, s_ref, b_ref, mask_ref, dsts, im_refs,
           *, sp, p, dpp, offsets, chunk, cin, relu):
    nb = len(srcs)
    cout = dsts[0].shape[0]
    zlead = jnp.zeros((cout, p + dpp), dsts[0].dtype)
    for dst in dsts:
        dst[:, 0:p + dpp] = zlead
        dst[:, p + sp - dpp:p + sp + p] = zlead
    for ci, base in enumerate(range(dpp, sp - dpp, chunk)):
        im_ref = im_refs[ci % len(im_refs)]
        for e, src in enumerate(srcs):
            for t, off in enumerate(offsets):
                im_ref[t * cin:(t + 1) * cin, e * chunk:(e + 1) * chunk] = \
                    src[:, off + base:off + base + chunk]
        acc = _dot_ta(w_ref[...], im_ref[...])
        res = acc * s_ref[...] + b_ref[...]
        if relu:
            res = jnp.maximum(res, 0.0)
        msk = mask_ref[:, base:base + chunk]
        for e, dst in enumerate(dsts):
            r = res[:, e * chunk:(e + 1) * chunk] * msk
            dst[:, p + base:p + base + chunk] = r.astype(dst.dtype)


def _make_c12_body(sp, p, dpp, offsets, chunk, c1_out):
    def _body(x_ref, w1_ref, w2_ref, s2_ref, b2_ref, mask_ref, o_ref,
              mid_ref, im1_ref, im_a_ref, im_b_ref):
        mid_ref[:, 0:p + dpp] = jnp.zeros((c1_out, p + dpp), mid_ref.dtype)
        mid_ref[:, p + sp - dpp:p + sp + p] = \
            jnp.zeros((c1_out, p + dpp), mid_ref.dtype)
        for base in range(dpp, sp - dpp, chunk):
            for t, off in enumerate(offsets):
                im1_ref[t:t + 1, :] = \
                    x_ref[:, off + base:off + base + chunk]
            acc = _dot_ta(w1_ref[...], im1_ref[...])
            nrm = jnp.sqrt(jnp.sum(acc * acc, axis=0, keepdims=True)) + 1e-9
            res = jnp.maximum(acc / nrm, 0.0) * mask_ref[:, base:base + chunk]
            mid_ref[:, p + base:p + base + chunk] = res.astype(mid_ref.dtype)
        _tconv([mid_ref], w2_ref, s2_ref, b2_ref, mask_ref, [o_ref.at[0]],
               [im_a_ref, im_b_ref], sp=sp, p=p, dpp=dpp, offsets=offsets,
               chunk=chunk, cin=c1_out, relu=False)
    return _body


def _conv12_t(x_flat, w1, w2, s2t, b2t, D):
    B = x_flat.shape[0]
    Dp = D + 2
    Sp = Dp ** 3
    P = Dp * Dp + Dp + 1
    Dpp = Dp * Dp
    chunk = _pick_chunk(Sp - 2 * Dpp)
    offsets = _tap_offsets(Dp, P)
    mask = _halo_mask_t(Dp)
    c1_out = w1.shape[-1]
    c2_out = w2.shape[-1]
    return pl.pallas_call(
        _make_c12_body(Sp, P, Dpp, offsets, chunk, c1_out),
        out_shape=jax.ShapeDtypeStruct((B, c2_out, Sp + 2 * P), jnp.bfloat16),
        grid=(B,),
        in_specs=[
            pl.BlockSpec((None, 1, Sp + 2 * P), lambda b: (b, 0, 0)),
            pl.BlockSpec((27, c1_out), lambda b: (0, 0)),
            pl.BlockSpec((27 * c1_out, c2_out), lambda b: (0, 0)),
            pl.BlockSpec((c2_out, 1), lambda b: (0, 0)),
            pl.BlockSpec((c2_out, 1), lambda b: (0, 0)),
            pl.BlockSpec((1, Sp), lambda b: (0, 0)),
        ],
        out_specs=pl.BlockSpec((1, c2_out, Sp + 2 * P), lambda b: (b, 0, 0)),
        scratch_shapes=[
            pltpu.VMEM((c1_out, Sp + 2 * P), jnp.bfloat16),
            pltpu.VMEM((27, chunk), jnp.bfloat16),
            pltpu.VMEM((27 * c1_out, chunk), jnp.bfloat16),
            pltpu.VMEM((27 * c1_out, chunk), jnp.bfloat16),
        ],
        compiler_params=pltpu.CompilerParams(
            dimension_semantics=("parallel",),
            vmem_limit_bytes=_VMEM_LIMIT),
    )(x_flat, w1, w2, s2t, b2t, mask)


def _make_pair_body(sp, p, dpp, offsets, chunk, nb, cin_a, cin_b):
    def _body(x_ref, wa_ref, sa_ref, ba_ref, wb_ref, sb_ref, bb_ref,
              mask_ref, o_ref, mid_ref, im_a_ref, im_b_ref):
        xs = [x_ref.at[e] for e in range(nb)]
        mids = [mid_ref.at[e] for e in range(nb)]
        outs = [o_ref.at[e] for e in range(nb)]
        _tconv(xs, wa_ref, sa_ref, ba_ref, mask_ref, mids, [im_a_ref],
               sp=sp, p=p, dpp=dpp, offsets=offsets, chunk=chunk,
               cin=cin_a, relu=True)
        _tconv(mids, wb_ref, sb_ref, bb_ref, mask_ref, outs, [im_b_ref],
               sp=sp, p=p, dpp=dpp, offsets=offsets, chunk=chunk,
               cin=cin_b, relu=True)
    return _body


def _conv_pair_t(x_t, pa, pb, D, nb):
    B, cin, total = x_t.shape
    Dp = D + 2
    Sp = Dp ** 3
    P = Dp * Dp + Dp + 1
    Dpp = Dp * Dp
    assert total == Sp + 2 * P and B % nb == 0
    wat, sat, bat = pa
    wbt, sbt, bbt = pb
    ca_out = wat.shape[-1]
    cb_out = wbt.shape[-1]
    chunk = Sp - 2 * Dpp
    offsets = _tap_offsets(Dp, P)
    mask = _halo_mask_t(Dp)
    return pl.pallas_call(
        _make_pair_body(Sp, P, Dpp, offsets, chunk, nb, cin, ca_out),
        out_shape=jax.ShapeDtypeStruct((B, cb_out, total), jnp.bfloat16),
        grid=(B // nb,),
        in_specs=[
            pl.BlockSpec((nb, cin, total), lambda b: (b, 0, 0)),
            pl.BlockSpec((27 * cin, ca_out), lambda b: (0, 0)),
            pl.BlockSpec((ca_out, 1), lambda b: (0, 0)),
            pl.BlockSpec((ca_out, 1), lambda b: (0, 0)),
            pl.BlockSpec((27 * ca_out, cb_out), lambda b: (0, 0)),
            pl.BlockSpec((cb_out, 1), lambda b: (0, 0)),
            pl.BlockSpec((cb_out, 1), lambda b: (0, 0)),
            pl.BlockSpec((1, Sp), lambda b: (0, 0)),
        ],
        out_specs=pl.BlockSpec((nb, cb_out, total), lambda b: (b, 0, 0)),
        scratch_shapes=[
            pltpu.VMEM((nb, ca_out, total), jnp.bfloat16),
            pltpu.VMEM((27 * cin, nb * chunk), jnp.bfloat16),
            pltpu.VMEM((27 * ca_out, nb * chunk), jnp.bfloat16),
        ],
        compiler_params=pltpu.CompilerParams(
            dimension_semantics=("parallel",),
            vmem_limit_bytes=_VMEM_LIMIT),
    )(x_t, wat, sat, bat, wbt, sbt, bbt, mask)


def _head_body(x_ref, w9_ref, s9_ref, b9_ref, w10_ref, s10_ref, b10_ref,
               w11_ref, s11_ref, b11_ref, o_ref):
    h = jnp.dot(x_ref[...], w9_ref[...], preferred_element_type=jnp.float32)
    h = jnp.maximum(h * s9_ref[...] + b9_ref[...], 0.0)
    h = jnp.dot(h.astype(jnp.bfloat16), w10_ref[...],
                preferred_element_type=jnp.float32)
    h = jnp.maximum(h * s10_ref[...] + b10_ref[...], 0.0)
    h = jnp.dot(h.astype(jnp.bfloat16), w11_ref[...],
                preferred_element_type=jnp.float32)
    h = jnp.maximum(h * s11_ref[...] + b11_ref[...], 0.0)
    z = h - jnp.max(h, axis=1, keepdims=True)
    e = jnp.exp(z)
    o_ref[...] = e / jnp.sum(e, axis=1, keepdims=True)


def _head(v, head_params):
    B, C = v.shape
    (w9, s9, b9), (w10, s10, b10), (w11, s11, b11) = head_params
    nc = w11.shape[-1]
    return pl.pallas_call(
        _head_body,
        out_shape=jax.ShapeDtypeStruct((B, nc), jnp.float32),
        in_specs=[pl.BlockSpec(v.shape, lambda: (0, 0))] +
                 [pl.BlockSpec(a.shape, lambda: (0, 0))
                  for a in (w9, s9, b9, w10, s10, b10, w11, s11, b11)],
        out_specs=pl.BlockSpec((B, nc), lambda: (0, 0)),
        compiler_params=pltpu.CompilerParams(
            vmem_limit_bytes=_VMEM_LIMIT),
    )(v, w9, s9, b9, w10, s10, b10, w11, s11, b11)


def _pool_pad_t(h_t, D):
    B, C = h_t.shape[0], h_t.shape[1]
    Dp = D + 2
    Sp = Dp ** 3
    P = Dp * Dp + Dp + 1
    Do = D // 2
    v = h_t[:, :, P:P + Sp]
    mw = jnp.maximum(v[:, :, :Sp - 1], v[:, :, 1:])
    mh = jnp.maximum(mw[:, :, :Sp - 1 - Dp], mw[:, :, Dp:])
    md = jnp.maximum(mh[:, :, :Sp - 1 - Dp - Dp * Dp], mh[:, :, Dp * Dp:])
    s0 = Dp * Dp + Dp + 1
    jlen = (Do - 1) * (Dp * Dp + Dp + 1) + 1
    t = md[:, :, s0:s0 + 2 * jlen:2]
    t = jnp.pad(t, ((0, 0), (0, 0), (0, Do * Dp * Dp - jlen)))
    pooled = t.reshape(B, C, Do, Dp, Dp)[:, :, :, :Do, :Do]
    if Do == 1:
        return pooled.reshape(B, C)
    Dq = Do + 2
    Pq = Dq * Dq + Dq + 1
    vp = jnp.pad(pooled, ((0, 0), (0, 0), (1, 1), (1, 1), (1, 1)))
    return jnp.pad(vp.reshape(B, C, Dq ** 3), ((0, 0), (0, 0), (Pq, Pq)))


@jax.jit
def _forward(x, params, head_params):
    B, D = x.shape[0], x.shape[1]
    xb = x.astype(jnp.bfloat16)
    Dp = D + 2
    Sp = Dp ** 3
    P = Dp * Dp + Dp + 1
    xp = jnp.pad(xb, ((0, 0), (1, 1), (1, 1), (1, 1)))
    x_flat = jnp.pad(xp.reshape(B, 1, Sp), ((0, 0), (0, 0), (P, P)))

    def tp(prm):
        w, s, b = prm
        return w, s.T, b.T

    h = _conv12_t(x_flat, params[0][0], params[1][0],
                  params[1][1].T, params[1][2].T, D)
    h = _conv_pair_t(_pool_pad_t(h, D), tp(params[2]), tp(params[3]),
                     D // 2, nb=math.gcd(2, B))
    h = _conv_pair_t(_pool_pad_t(h, D // 2), tp(params[4]), tp(params[5]),
                     D // 4, nb=math.gcd(8, B))
    h = _conv_pair_t(_pool_pad_t(h, D // 4), tp(params[6]), tp(params[7]),
                     D // 8, nb=math.gcd(16, B))
    v = _pool_pad_t(h, D // 8)
    return _head(v, head_params)


def kernel(x, w0, s0, sh0, w1, s1, sh1, w2, s2, sh2, w3, s3, sh3,
           w4, s4, sh4, w5, s5, sh5, w6, s6, sh6, w7, s7, sh7,
           w8, s8, sh8, w9, s9, sh9, w10, s10, sh10,
           hw0, hs0, hb0, hw1, hs1, hb1, hw2, hs2, hb2):
    params = [(w0, s0, sh0), (w1, s1, sh1), (w2, s2, sh2), (w3, s3, sh3),
              (w4, s4, sh4), (w5, s5, sh5), (w6, s6, sh6), (w7, s7, sh7)]
    head_params = ((hw0, hs0, hb0), (hw1, hs1, hb1), (hw2, hs2, hb2))
    return _forward(x, params, head_params)


# speedup vs baseline: 1.6167x; 1.6167x over previous
import math

import numpy as np
import jax
import jax.numpy as jnp
from jax.experimental import pallas as pl
from jax.experimental.pallas import tpu as pltpu


_VMEM_LIMIT = 48 * 1024 * 1024


def _tap_offsets(dp, p):
    return tuple(p + (kd - 1) * dp * dp + (kh - 1) * dp + (kw - 1)
                 for kd in range(3) for kh in range(3) for kw in range(3))


def _halo_mask_t(dp):
    m = np.zeros((dp, dp, dp), np.float32)
    m[1:-1, 1:-1, 1:-1] = 1.0
    return jnp.asarray(m.reshape(1, dp * dp * dp))


def _pick_chunk(sp, cap=2048):
    if sp <= cap:
        return sp
    for n in range(2, sp + 1):
        if sp % n == 0 and sp // n <= cap:
            return sp // n
    return sp


def _dot_ta(w, im):
    return jax.lax.dot_general(w, im, (((0,), (0,)), ((), ())),
                               preferred_element_type=jnp.float32)


def _tconv(srcs, w_ref, s_ref, b_ref, mask_ref, dsts, im_refs,
           *, sp, p, dpp, offsets, chunk, cin, relu):
    nb = len(srcs)
    cout = dsts[0].shape[0]
    zlead = jnp.zeros((cout, p + dpp), dsts[0].dtype)
    for dst in dsts:
        dst[:, 0:p + dpp] = zlead
        dst[:, p + sp - dpp:p + sp + p] = zlead
    for ci, base in enumerate(range(dpp, sp - dpp, chunk)):
        im_ref = im_refs[ci % len(im_refs)]
        for e, src in enumerate(srcs):
            for t, off in enumerate(offsets):
                im_ref[t * cin:(t + 1) * cin, e * chunk:(e + 1) * chunk] = \
                    src[:, off + base:off + base + chunk]
        acc = _dot_ta(w_ref[...], im_ref[...])
        res = acc * s_ref[...] + b_ref[...]
        if relu:
            res = jnp.maximum(res, 0.0)
        msk = mask_ref[:, base:base + chunk]
        for e, dst in enumerate(dsts):
            r = res[:, e * chunk:(e + 1) * chunk] * msk
            dst[:, p + base:p + base + chunk] = r.astype(dst.dtype)


def _make_c12_body(sp, p, dpp, offsets, chunk, c1_out):
    def _body(x_ref, w1_ref, w2_ref, s2_ref, b2_ref, mask_ref, o_ref,
              mid_ref, im1_ref, im_a_ref, im_b_ref):
        mid_ref[:, 0:p + dpp] = jnp.zeros((c1_out, p + dpp), mid_ref.dtype)
        mid_ref[:, p + sp - dpp:p + sp + p] = \
            jnp.zeros((c1_out, p + dpp), mid_ref.dtype)
        for base in range(dpp, sp - dpp, chunk):
            for t, off in enumerate(offsets):
                im1_ref[t:t + 1, :] = \
                    x_ref[:, off + base:off + base + chunk]
            acc = _dot_ta(w1_ref[...], im1_ref[...])
            nrm = jnp.sqrt(jnp.sum(acc * acc, axis=0, keepdims=True)) + 1e-9
            res = jnp.maximum(acc / nrm, 0.0) * mask_ref[:, base:base + chunk]
            mid_ref[:, p + base:p + base + chunk] = res.astype(mid_ref.dtype)
        _tconv([mid_ref], w2_ref, s2_ref, b2_ref, mask_ref, [o_ref.at[0]],
               [im_a_ref, im_b_ref], sp=sp, p=p, dpp=dpp, offsets=offsets,
               chunk=chunk, cin=c1_out, relu=False)
    return _body


def _conv12_t(x_flat, w1, w2, s2t, b2t, D):
    B = x_flat.shape[0]
    Dp = D + 2
    Sp = Dp ** 3
    P = Dp * Dp + Dp + 1
    Dpp = Dp * Dp
    chunk = _pick_chunk(Sp - 2 * Dpp)
    offsets = _tap_offsets(Dp, P)
    mask = _halo_mask_t(Dp)
    c1_out = w1.shape[-1]
    c2_out = w2.shape[-1]
    return pl.pallas_call(
        _make_c12_body(Sp, P, Dpp, offsets, chunk, c1_out),
        out_shape=jax.ShapeDtypeStruct((B, c2_out, Sp + 2 * P), jnp.bfloat16),
        grid=(B,),
        in_specs=[
            pl.BlockSpec((None, 1, Sp + 2 * P), lambda b: (b, 0, 0)),
            pl.BlockSpec((27, c1_out), lambda b: (0, 0)),
            pl.BlockSpec((27 * c1_out, c2_out), lambda b: (0, 0)),
            pl.BlockSpec((c2_out, 1), lambda b: (0, 0)),
            pl.BlockSpec((c2_out, 1), lambda b: (0, 0)),
            pl.BlockSpec((1, Sp), lambda b: (0, 0)),
        ],
        out_specs=pl.BlockSpec((1, c2_out, Sp + 2 * P), lambda b: (b, 0, 0)),
        scratch_shapes=[
            pltpu.VMEM((c1_out, Sp + 2 * P), jnp.bfloat16),
            pltpu.VMEM((27, chunk), jnp.bfloat16),
            pltpu.VMEM((27 * c1_out, chunk), jnp.bfloat16),
            pltpu.VMEM((27 * c1_out, chunk), jnp.bfloat16),
        ],
        compiler_params=pltpu.CompilerParams(
            dimension_semantics=("parallel",),
            vmem_limit_bytes=_VMEM_LIMIT),
    )(x_flat, w1, w2, s2t, b2t, mask)


def _make_pair_body(sp, p, dpp, offsets, chunk, nb, cin_a, cin_b):
    def _body(x_ref, wa_ref, sa_ref, ba_ref, wb_ref, sb_ref, bb_ref,
              mask_ref, o_ref, mid_ref, im_a_ref, im_b_ref):
        xs = [x_ref.at[e] for e in range(nb)]
        mids = [mid_ref.at[e] for e in range(nb)]
        outs = [o_ref.at[e] for e in range(nb)]
        _tconv(xs, wa_ref, sa_ref, ba_ref, mask_ref, mids, [im_a_ref],
               sp=sp, p=p, dpp=dpp, offsets=offsets, chunk=chunk,
               cin=cin_a, relu=True)
        _tconv(mids, wb_ref, sb_ref, bb_ref, mask_ref, outs, [im_b_ref],
               sp=sp, p=p, dpp=dpp, offsets=offsets, chunk=chunk,
               cin=cin_b, relu=True)
    return _body


def _conv_pair_t(x_t, pa, pb, D, nb):
    B, cin, total = x_t.shape
    Dp = D + 2
    Sp = Dp ** 3
    P = Dp * Dp + Dp + 1
    Dpp = Dp * Dp
    assert total == Sp + 2 * P and B % nb == 0
    wat, sat, bat = pa
    wbt, sbt, bbt = pb
    ca_out = wat.shape[-1]
    cb_out = wbt.shape[-1]
    chunk = Sp - 2 * Dpp
    offsets = _tap_offsets(Dp, P)
    mask = _halo_mask_t(Dp)
    return pl.pallas_call(
        _make_pair_body(Sp, P, Dpp, offsets, chunk, nb, cin, ca_out),
        out_shape=jax.ShapeDtypeStruct((B, cb_out, total), jnp.bfloat16),
        grid=(B // nb,),
        in_specs=[
            pl.BlockSpec((nb, cin, total), lambda b: (b, 0, 0)),
            pl.BlockSpec((27 * cin, ca_out), lambda b: (0, 0)),
            pl.BlockSpec((ca_out, 1), lambda b: (0, 0)),
            pl.BlockSpec((ca_out, 1), lambda b: (0, 0)),
            pl.BlockSpec((27 * ca_out, cb_out), lambda b: (0, 0)),
            pl.BlockSpec((cb_out, 1), lambda b: (0, 0)),
            pl.BlockSpec((cb_out, 1), lambda b: (0, 0)),
            pl.BlockSpec((1, Sp), lambda b: (0, 0)),
        ],
        out_specs=pl.BlockSpec((nb, cb_out, total), lambda b: (b, 0, 0)),
        scratch_shapes=[
            pltpu.VMEM((nb, ca_out, total), jnp.bfloat16),
            pltpu.VMEM((27 * cin, nb * chunk), jnp.bfloat16),
            pltpu.VMEM((27 * ca_out, nb * chunk), jnp.bfloat16),
        ],
        compiler_params=pltpu.CompilerParams(
            dimension_semantics=("parallel",),
            vmem_limit_bytes=_VMEM_LIMIT),
    )(x_t, wat, sat, bat, wbt, sbt, bbt, mask)


def _head_body(x_ref, w9_ref, s9_ref, b9_ref, w10_ref, s10_ref, b10_ref,
               w11_ref, s11_ref, b11_ref, o_ref):
    h = jnp.dot(x_ref[...], w9_ref[...], preferred_element_type=jnp.float32)
    h = jnp.maximum(h * s9_ref[...] + b9_ref[...], 0.0)
    h = jnp.dot(h.astype(jnp.bfloat16), w10_ref[...],
                preferred_element_type=jnp.float32)
    h = jnp.maximum(h * s10_ref[...] + b10_ref[...], 0.0)
    h = jnp.dot(h.astype(jnp.bfloat16), w11_ref[...],
                preferred_element_type=jnp.float32)
    h = jnp.maximum(h * s11_ref[...] + b11_ref[...], 0.0)
    z = h - jnp.max(h, axis=1, keepdims=True)
    e = jnp.exp(z)
    o_ref[...] = e / jnp.sum(e, axis=1, keepdims=True)


def _head(v, head_params):
    B, C = v.shape
    (w9, s9, b9), (w10, s10, b10), (w11, s11, b11) = head_params
    nc = w11.shape[-1]
    return pl.pallas_call(
        _head_body,
        out_shape=jax.ShapeDtypeStruct((B, nc), jnp.float32),
        in_specs=[pl.BlockSpec(v.shape, lambda: (0, 0))] +
                 [pl.BlockSpec(a.shape, lambda: (0, 0))
                  for a in (w9, s9, b9, w10, s10, b10, w11, s11, b11)],
        out_specs=pl.BlockSpec((B, nc), lambda: (0, 0)),
        compiler_params=pltpu.CompilerParams(
            vmem_limit_bytes=_VMEM_LIMIT),
    )(v, w9, s9, b9, w10, s10, b10, w11, s11, b11)


def _pool_pad_t(h_t, D):
    B, C = h_t.shape[0], h_t.shape[1]
    Dp = D + 2
    Sp = Dp ** 3
    P = Dp * Dp + Dp + 1
    Do = D // 2
    v = h_t[:, :, P:P + Sp].reshape(B, C, Dp, Dp, Dp)
    v = v[:, :, 1:1 + D, 1:1 + D, 1:1 + D]
    pooled = v.reshape(B, C, Do, 2, Do, 2, Do, 2).max(axis=(3, 5, 7))
    if Do == 1:
        return pooled.reshape(B, C)
    Dq = Do + 2
    Pq = Dq * Dq + Dq + 1
    vp = jnp.pad(pooled, ((0, 0), (0, 0), (1, 1), (1, 1), (1, 1)))
    return jnp.pad(vp.reshape(B, C, Dq ** 3), ((0, 0), (0, 0), (Pq, Pq)))


@jax.jit
def _forward(x, params, head_params):
    B, D = x.shape[0], x.shape[1]
    xb = x.astype(jnp.bfloat16)
    Dp = D + 2
    Sp = Dp ** 3
    P = Dp * Dp + Dp + 1
    xp = jnp.pad(xb, ((0, 0), (1, 1), (1, 1), (1, 1)))
    x_flat = jnp.pad(xp.reshape(B, 1, Sp), ((0, 0), (0, 0), (P, P)))

    def tp(prm):
        w, s, b = prm
        return w, s.T, b.T

    h = _conv12_t(x_flat, params[0][0], params[1][0],
                  params[1][1].T, params[1][2].T, D)
    h = _conv_pair_t(_pool_pad_t(h, D), tp(params[2]), tp(params[3]),
                     D // 2, nb=math.gcd(2, B))
    h = _conv_pair_t(_pool_pad_t(h, D // 2), tp(params[4]), tp(params[5]),
                     D // 4, nb=math.gcd(8, B))
    h = _conv_pair_t(_pool_pad_t(h, D // 4), tp(params[6]), tp(params[7]),
                     D // 8, nb=math.gcd(16, B))
    v = _pool_pad_t(h, D // 8)
    return _head(v, head_params)


def kernel(x, w0, s0, sh0, w1, s1, sh1, w2, s2, sh2, w3, s3, sh3,
           w4, s4, sh4, w5, s5, sh5, w6, s6, sh6, w7, s7, sh7,
           w8, s8, sh8, w9, s9, sh9, w10, s10, sh10,
           hw0, hs0, hb0, hw1, hs1, hb1, hw2, hs2, hb2):
    params = [(w0, s0, sh0), (w1, s1, sh1), (w2, s2, sh2), (w3, s3, sh3),
              (w4, s4, sh4), (w5, s5, sh5), (w6, s6, sh6), (w7, s7, sh7)]
    head_params = ((hw0, hs0, hb0), (hw1, hs1, hb1), (hw2, hs2, hb2))
    return _forward(x, params, head_params)


# speedup vs baseline: 1.6213x; 1.0028x over previous
import math

import numpy as np
import jax
import jax.numpy as jnp
from jax.experimental import pallas as pl
from jax.experimental.pallas import tpu as pltpu


_VMEM_LIMIT = 48 * 1024 * 1024


def _tap_offsets(dp, p):
    return tuple(p + (kd - 1) * dp * dp + (kh - 1) * dp + (kw - 1)
                 for kd in range(3) for kh in range(3) for kw in range(3))


def _halo_mask_t(dp):
    m = np.zeros((dp, dp, dp), np.float32)
    m[1:-1, 1:-1, 1:-1] = 1.0
    return jnp.asarray(m.reshape(1, dp * dp * dp))


def _pick_chunk(sp, cap=2048):
    if sp <= cap:
        return sp
    for n in range(2, sp + 1):
        if sp % n == 0 and sp // n <= cap:
            return sp // n
    return sp


def _dot_ta(w, im):
    return jax.lax.dot_general(w, im, (((0,), (0,)), ((), ())),
                               preferred_element_type=jnp.float32)


def _tconv(srcs, w_ref, s_ref, b_ref, mask_ref, dsts, im_refs,
           *, sp, p, dpp, offsets, chunk, cin, relu):
    nb = len(srcs)
    cout = dsts[0].shape[0]
    zlead = jnp.zeros((cout, p + dpp), dsts[0].dtype)
    for dst in dsts:
        dst[:, 0:p + dpp] = zlead
        dst[:, p + sp - dpp:p + sp + p] = zlead
    for ci, base in enumerate(range(dpp, sp - dpp, chunk)):
        im_ref = im_refs[ci % len(im_refs)]
        for e, src in enumerate(srcs):
            for t, off in enumerate(offsets):
                im_ref[t * cin:(t + 1) * cin, e * chunk:(e + 1) * chunk] = \
                    src[:, off + base:off + base + chunk]
        acc = _dot_ta(w_ref[...], im_ref[...])
        res = acc * s_ref[...] + b_ref[...]
        if relu:
            res = jnp.maximum(res, 0.0)
        msk = mask_ref[:, base:base + chunk]
        for e, dst in enumerate(dsts):
            r = res[:, e * chunk:(e + 1) * chunk] * msk
            dst[:, p + base:p + base + chunk] = r.astype(dst.dtype)


def _pool_sel(dp):
    d = dp - 2
    do = d // 2
    dq = do + 2
    sel = np.zeros((dp * dp, dq * dq), np.float32)
    for oh in range(do):
        for ow in range(do):
            sel[(1 + 2 * oh) * dp + (1 + 2 * ow), (1 + oh) * dq + (1 + ow)] = 1.0
    return jnp.asarray(sel, jnp.bfloat16)


def _pool_write(src_ref, out_ref, sel_ref, *, sp, p, dp):
    dpp = dp * dp
    do = (dp - 2) // 2
    if do == 1:
        W = src_ref[:, p:p + sp]
        mw = jnp.maximum(W[:, :sp - 1], W[:, 1:sp])
        mh = jnp.maximum(mw[:, :sp - 1 - dp], mw[:, dp:sp - 1])
        md = jnp.maximum(mh[:, :sp - 1 - dp - dpp], mh[:, dpp:sp - 1 - dp])
        s0 = dpp + dp + 1
        out_ref[:, 0:1] = md[:, s0:s0 + 1]
        return
    dq = do + 2
    dq2 = dq * dq
    sq = dq ** 3
    pq = dq2 + dq + 1
    cout = out_ref.shape[0]
    out_ref[:, 0:pq + dq2] = jnp.zeros((cout, pq + dq2), out_ref.dtype)
    out_ref[:, pq + sq - dq2:sq + 2 * pq] = \
        jnp.zeros((cout, pq + dq2), out_ref.dtype)
    L = 2 * dpp + dp + 2
    for od in range(do):
        a = p + (1 + 2 * od) * dpp
        W = src_ref[:, a:a + L]
        mw = jnp.maximum(W[:, :L - 1], W[:, 1:L])
        mh = jnp.maximum(mw[:, :L - 1 - dp], mw[:, dp:L - 1])
        md = jnp.maximum(mh[:, :dpp], mh[:, dpp:2 * dpp])
        r = jnp.dot(md, sel_ref[...], preferred_element_type=jnp.float32)
        out_ref[:, pq + (1 + od) * dq2:pq + (2 + od) * dq2] = \
            r.astype(out_ref.dtype)


def _make_c12_body(sp, p, dpp, offsets, chunk, c1_out):
    dp = round(dpp ** 0.5)

    def _body(x_ref, w1_ref, w2_ref, s2_ref, b2_ref, mask_ref, sel_ref,
              o_ref, mid_ref, act2_ref, im1_ref, im_a_ref, im_b_ref):
        mid_ref[:, 0:p + dpp] = jnp.zeros((c1_out, p + dpp), mid_ref.dtype)
        mid_ref[:, p + sp - dpp:p + sp + p] = \
            jnp.zeros((c1_out, p + dpp), mid_ref.dtype)
        for base in range(dpp, sp - dpp, chunk):
            for t, off in enumerate(offsets):
                im1_ref[t:t + 1, :] = \
                    x_ref[:, off + base:off + base + chunk]
            acc = _dot_ta(w1_ref[...], im1_ref[...])
            nrm = jnp.sqrt(jnp.sum(acc * acc, axis=0, keepdims=True)) + 1e-9
            res = jnp.maximum(acc / nrm, 0.0) * mask_ref[:, base:base + chunk]
            mid_ref[:, p + base:p + base + chunk] = res.astype(mid_ref.dtype)
        _tconv([mid_ref], w2_ref, s2_ref, b2_ref, mask_ref, [act2_ref],
               [im_a_ref, im_b_ref], sp=sp, p=p, dpp=dpp, offsets=offsets,
               chunk=chunk, cin=c1_out, relu=False)
        _pool_write(act2_ref, o_ref.at[0], sel_ref, sp=sp, p=p, dp=dp)
    return _body


def _conv12_t(x_flat, w1, w2, s2t, b2t, D):
    B = x_flat.shape[0]
    Dp = D + 2
    Sp = Dp ** 3
    P = Dp * Dp + Dp + 1
    Dpp = Dp * Dp
    chunk = _pick_chunk(Sp - 2 * Dpp)
    offsets = _tap_offsets(Dp, P)
    mask = _halo_mask_t(Dp)
    c1_out = w1.shape[-1]
    c2_out = w2.shape[-1]
    Dq = D // 2 + 2
    Pq = Dq * Dq + Dq + 1
    total_q = Dq ** 3 + 2 * Pq
    return pl.pallas_call(
        _make_c12_body(Sp, P, Dpp, offsets, chunk, c1_out),
        out_shape=jax.ShapeDtypeStruct((B, c2_out, total_q), jnp.bfloat16),
        grid=(B,),
        in_specs=[
            pl.BlockSpec((None, 1, Sp + 2 * P), lambda b: (b, 0, 0)),
            pl.BlockSpec((27, c1_out), lambda b: (0, 0)),
            pl.BlockSpec((27 * c1_out, c2_out), lambda b: (0, 0)),
            pl.BlockSpec((c2_out, 1), lambda b: (0, 0)),
            pl.BlockSpec((c2_out, 1), lambda b: (0, 0)),
            pl.BlockSpec((1, Sp), lambda b: (0, 0)),
            pl.BlockSpec((Dpp, Dq * Dq), lambda b: (0, 0)),
        ],
        out_specs=pl.BlockSpec((1, c2_out, total_q), lambda b: (b, 0, 0)),
        scratch_shapes=[
            pltpu.VMEM((c1_out, Sp + 2 * P), jnp.bfloat16),
            pltpu.VMEM((c2_out, Sp + 2 * P), jnp.bfloat16),
            pltpu.VMEM((27, chunk), jnp.bfloat16),
            pltpu.VMEM((27 * c1_out, chunk), jnp.bfloat16),
            pltpu.VMEM((27 * c1_out, chunk), jnp.bfloat16),
        ],
        compiler_params=pltpu.CompilerParams(
            dimension_semantics=("parallel",),
            vmem_limit_bytes=_VMEM_LIMIT),
    )(x_flat, w1, w2, s2t, b2t, mask, _pool_sel(Dp))


def _make_pair_body(sp, p, dpp, offsets, chunk, nb, cin_a, cin_b):
    dp = round(dpp ** 0.5)

    def _body(x_ref, wa_ref, sa_ref, ba_ref, wb_ref, sb_ref, bb_ref,
              mask_ref, sel_ref, o_ref, mid_ref, act_ref, im_a_ref, im_b_ref):
        xs = [x_ref.at[e] for e in range(nb)]
        mids = [mid_ref.at[e] for e in range(nb)]
        acts = [act_ref.at[e] for e in range(nb)]
        _tconv(xs, wa_ref, sa_ref, ba_ref, mask_ref, mids, [im_a_ref],
               sp=sp, p=p, dpp=dpp, offsets=offsets, chunk=chunk,
               cin=cin_a, relu=True)
        _tconv(mids, wb_ref, sb_ref, bb_ref, mask_ref, acts, [im_b_ref],
               sp=sp, p=p, dpp=dpp, offsets=offsets, chunk=chunk,
               cin=cin_b, relu=True)
        for e in range(nb):
            _pool_write(act_ref.at[e], o_ref.at[e], sel_ref, sp=sp, p=p, dp=dp)
    return _body


def _conv_pair_t(x_t, pa, pb, D, nb):
    B, cin, total = x_t.shape
    Dp = D + 2
    Sp = Dp ** 3
    P = Dp * Dp + Dp + 1
    Dpp = Dp * Dp
    assert total == Sp + 2 * P and B % nb == 0
    wat, sat, bat = pa
    wbt, sbt, bbt = pb
    ca_out = wat.shape[-1]
    cb_out = wbt.shape[-1]
    chunk = Sp - 2 * Dpp
    offsets = _tap_offsets(Dp, P)
    mask = _halo_mask_t(Dp)
    Do = D // 2
    if Do == 1:
        total_q = 1
    else:
        Dq = Do + 2
        total_q = Dq ** 3 + 2 * (Dq * Dq + Dq + 1)
    return pl.pallas_call(
        _make_pair_body(Sp, P, Dpp, offsets, chunk, nb, cin, ca_out),
        out_shape=jax.ShapeDtypeStruct((B, cb_out, total_q), jnp.bfloat16),
        grid=(B // nb,),
        in_specs=[
            pl.BlockSpec((nb, cin, total), lambda b: (b, 0, 0)),
            pl.BlockSpec((27 * cin, ca_out), lambda b: (0, 0)),
            pl.BlockSpec((ca_out, 1), lambda b: (0, 0)),
            pl.BlockSpec((ca_out, 1), lambda b: (0, 0)),
            pl.BlockSpec((27 * ca_out, cb_out), lambda b: (0, 0)),
            pl.BlockSpec((cb_out, 1), lambda b: (0, 0)),
            pl.BlockSpec((cb_out, 1), lambda b: (0, 0)),
            pl.BlockSpec((1, Sp), lambda b: (0, 0)),
            pl.BlockSpec((Dpp, (Do + 2) ** 2), lambda b: (0, 0)),
        ],
        out_specs=pl.BlockSpec((nb, cb_out, total_q), lambda b: (b, 0, 0)),
        scratch_shapes=[
            pltpu.VMEM((nb, ca_out, total), jnp.bfloat16),
            pltpu.VMEM((nb, cb_out, total), jnp.bfloat16),
            pltpu.VMEM((27 * cin, nb * chunk), jnp.bfloat16),
            pltpu.VMEM((27 * ca_out, nb * chunk), jnp.bfloat16),
        ],
        compiler_params=pltpu.CompilerParams(
            dimension_semantics=("parallel",),
            vmem_limit_bytes=_VMEM_LIMIT),
    )(x_t, wat, sat, bat, wbt, sbt, bbt, mask, _pool_sel(Dp))


def _head_body(x_ref, w9_ref, s9_ref, b9_ref, w10_ref, s10_ref, b10_ref,
               w11_ref, s11_ref, b11_ref, o_ref):
    h = jnp.dot(x_ref[...], w9_ref[...], preferred_element_type=jnp.float32)
    h = jnp.maximum(h * s9_ref[...] + b9_ref[...], 0.0)
    h = jnp.dot(h.astype(jnp.bfloat16), w10_ref[...],
                preferred_element_type=jnp.float32)
    h = jnp.maximum(h * s10_ref[...] + b10_ref[...], 0.0)
    h = jnp.dot(h.astype(jnp.bfloat16), w11_ref[...],
                preferred_element_type=jnp.float32)
    h = jnp.maximum(h * s11_ref[...] + b11_ref[...], 0.0)
    z = h - jnp.max(h, axis=1, keepdims=True)
    e = jnp.exp(z)
    o_ref[...] = e / jnp.sum(e, axis=1, keepdims=True)


def _head(v, head_params):
    B, C = v.shape
    (w9, s9, b9), (w10, s10, b10), (w11, s11, b11) = head_params
    nc = w11.shape[-1]
    return pl.pallas_call(
        _head_body,
        out_shape=jax.ShapeDtypeStruct((B, nc), jnp.float32),
        in_specs=[pl.BlockSpec(v.shape, lambda: (0, 0))] +
                 [pl.BlockSpec(a.shape, lambda: (0, 0))
                  for a in (w9, s9, b9, w10, s10, b10, w11, s11, b11)],
        out_specs=pl.BlockSpec((B, nc), lambda: (0, 0)),
        compiler_params=pltpu.CompilerParams(
            vmem_limit_bytes=_VMEM_LIMIT),
    )(v, w9, s9, b9, w10, s10, b10, w11, s11, b11)


@jax.jit
def _forward(x, params, head_params):
    B, D = x.shape[0], x.shape[1]
    xb = x.astype(jnp.bfloat16)
    Dp = D + 2
    Sp = Dp ** 3
    P = Dp * Dp + Dp + 1
    xp = jnp.pad(xb, ((0, 0), (1, 1), (1, 1), (1, 1)))
    x_flat = jnp.pad(xp.reshape(B, 1, Sp), ((0, 0), (0, 0), (P, P)))

    def tp(prm):
        w, s, b = prm
        return w, s.T, b.T

    h = _conv12_t(x_flat, params[0][0], params[1][0],
                  params[1][1].T, params[1][2].T, D)
    h = _conv_pair_t(h, tp(params[2]), tp(params[3]), D // 2,
                     nb=math.gcd(2, B))
    h = _conv_pair_t(h, tp(params[4]), tp(params[5]), D // 4,
                     nb=math.gcd(8, B))
    h = _conv_pair_t(h, tp(params[6]), tp(params[7]), D // 8,
                     nb=math.gcd(16, B))
    return _head(h.reshape(B, 256), head_params)


def kernel(x, w0, s0, sh0, w1, s1, sh1, w2, s2, sh2, w3, s3, sh3,
           w4, s4, sh4, w5, s5, sh5, w6, s6, sh6, w7, s7, sh7,
           w8, s8, sh8, w9, s9, sh9, w10, s10, sh10,
           hw0, hs0, hb0, hw1, hs1, hb1, hw2, hs2, hb2):
    params = [(w0, s0, sh0), (w1, s1, sh1), (w2, s2, sh2), (w3, s3, sh3),
              (w4, s4, sh4), (w5, s5, sh5), (w6, s6, sh6), (w7, s7, sh7)]
    head_params = ((hw0, hs0, hb0), (hw1, hs1, hb1), (hw2, hs2, hb2))
    return _forward(x, params, head_params)


# speedup vs baseline: 1.6334x; 1.0075x over previous
import math

import numpy as np
import jax
import jax.numpy as jnp
from jax.experimental import pallas as pl
from jax.experimental.pallas import tpu as pltpu


_VMEM_LIMIT = 48 * 1024 * 1024


def _tap_offsets(dp, p):
    return tuple(p + (kd - 1) * dp * dp + (kh - 1) * dp + (kw - 1)
                 for kd in range(3) for kh in range(3) for kw in range(3))


def _halo_mask_t(dp):
    m = np.zeros((dp, dp, dp), np.float32)
    m[1:-1, 1:-1, 1:-1] = 1.0
    return jnp.asarray(m.reshape(1, dp * dp * dp))


def _pick_chunk(sp, cap=2048):
    if sp <= cap:
        return sp
    for n in range(2, sp + 1):
        if sp % n == 0 and sp // n <= cap:
            return sp // n
    return sp


def _dot_ta(w, im):
    return jax.lax.dot_general(w, im, (((0,), (0,)), ((), ())),
                               preferred_element_type=jnp.float32)


def _tconv(srcs, w_ref, s_ref, b_ref, mask_ref, dsts, im_refs,
           *, sp, p, dpp, offsets, chunk, cin, relu, masked=True):
    nb = len(srcs)
    cout = dsts[0].shape[0]
    if masked:
        zlead = jnp.zeros((cout, p + dpp), dsts[0].dtype)
        for dst in dsts:
            dst[:, 0:p + dpp] = zlead
            dst[:, p + sp - dpp:p + sp + p] = zlead
    for ci, base in enumerate(range(dpp, sp - dpp, chunk)):
        im_ref = im_refs[ci % len(im_refs)]
        for e, src in enumerate(srcs):
            for t, off in enumerate(offsets):
                im_ref[t * cin:(t + 1) * cin, e * chunk:(e + 1) * chunk] = \
                    src[:, off + base:off + base + chunk]
        acc = _dot_ta(w_ref[...], im_ref[...])
        res = acc * s_ref[...] + b_ref[...]
        if relu:
            res = jnp.maximum(res, 0.0)
        msk = mask_ref[:, base:base + chunk] if masked else None
        for e, dst in enumerate(dsts):
            r = res[:, e * chunk:(e + 1) * chunk]
            if masked:
                r = r * msk
            dst[:, p + base:p + base + chunk] = r.astype(dst.dtype)


def _pool_sel(dp):
    d = dp - 2
    do = d // 2
    dq = do + 2
    sel = np.zeros((dp * dp, dq * dq), np.float32)
    for oh in range(do):
        for ow in range(do):
            sel[(1 + 2 * oh) * dp + (1 + 2 * ow), (1 + oh) * dq + (1 + ow)] = 1.0
    return jnp.asarray(sel, jnp.bfloat16)


def _pool_write(src_ref, out_ref, sel_ref, *, sp, p, dp):
    dpp = dp * dp
    do = (dp - 2) // 2
    if do == 1:
        W = src_ref[:, p:p + sp]
        mw = jnp.maximum(W[:, :sp - 1], W[:, 1:sp])
        mh = jnp.maximum(mw[:, :sp - 1 - dp], mw[:, dp:sp - 1])
        md = jnp.maximum(mh[:, :sp - 1 - dp - dpp], mh[:, dpp:sp - 1 - dp])
        s0 = dpp + dp + 1
        out_ref[:, 0:1] = md[:, s0:s0 + 1]
        return
    dq = do + 2
    dq2 = dq * dq
    sq = dq ** 3
    pq = dq2 + dq + 1
    cout = out_ref.shape[0]
    out_ref[:, 0:pq + dq2] = jnp.zeros((cout, pq + dq2), out_ref.dtype)
    out_ref[:, pq + sq - dq2:sq + 2 * pq] = \
        jnp.zeros((cout, pq + dq2), out_ref.dtype)
    L = 2 * dpp + dp + 2
    for od in range(do):
        a = p + (1 + 2 * od) * dpp
        W = src_ref[:, a:a + L]
        mw = jnp.maximum(W[:, :L - 1], W[:, 1:L])
        mh = jnp.maximum(mw[:, :L - 1 - dp], mw[:, dp:L - 1])
        md = jnp.maximum(mh[:, :dpp], mh[:, dpp:2 * dpp])
        r = jnp.dot(md, sel_ref[...], preferred_element_type=jnp.float32)
        out_ref[:, pq + (1 + od) * dq2:pq + (2 + od) * dq2] = \
            r.astype(out_ref.dtype)


def _make_c12_body(sp, p, dpp, offsets, chunk, c1_out):
    dp = round(dpp ** 0.5)

    def _body(x_ref, w1_ref, w2_ref, s2_ref, b2_ref, mask_ref, sel_ref,
              o_ref, mid_ref, act2_ref, im1_ref, im_a_ref, im_b_ref):
        mid_ref[:, 0:p + dpp] = jnp.zeros((c1_out, p + dpp), mid_ref.dtype)
        mid_ref[:, p + sp - dpp:p + sp + p] = \
            jnp.zeros((c1_out, p + dpp), mid_ref.dtype)
        for base in range(dpp, sp - dpp, chunk):
            for t, off in enumerate(offsets):
                im1_ref[t:t + 1, :] = \
                    x_ref[:, off + base:off + base + chunk]
            acc = _dot_ta(w1_ref[...], im1_ref[...])
            nrm = jnp.sqrt(jnp.sum(acc * acc, axis=0, keepdims=True)) + 1e-9
            res = jnp.maximum(acc / nrm, 0.0) * mask_ref[:, base:base + chunk]
            mid_ref[:, p + base:p + base + chunk] = res.astype(mid_ref.dtype)
        _tconv([mid_ref], w2_ref, s2_ref, b2_ref, mask_ref, [act2_ref],
               [im_a_ref, im_b_ref], sp=sp, p=p, dpp=dpp, offsets=offsets,
               chunk=chunk, cin=c1_out, relu=False, masked=False)
        _pool_write(act2_ref, o_ref.at[0], sel_ref, sp=sp, p=p, dp=dp)
    return _body


def _conv12_t(x_flat, w1, w2, s2t, b2t, D):
    B = x_flat.shape[0]
    Dp = D + 2
    Sp = Dp ** 3
    P = Dp * Dp + Dp + 1
    Dpp = Dp * Dp
    chunk = _pick_chunk(Sp - 2 * Dpp)
    offsets = _tap_offsets(Dp, P)
    mask = _halo_mask_t(Dp)
    c1_out = w1.shape[-1]
    c2_out = w2.shape[-1]
    Dq = D // 2 + 2
    Pq = Dq * Dq + Dq + 1
    total_q = Dq ** 3 + 2 * Pq
    return pl.pallas_call(
        _make_c12_body(Sp, P, Dpp, offsets, chunk, c1_out),
        out_shape=jax.ShapeDtypeStruct((B, c2_out, total_q), jnp.bfloat16),
        grid=(B,),
        in_specs=[
            pl.BlockSpec((None, 1, Sp + 2 * P), lambda b: (b, 0, 0)),
            pl.BlockSpec((27, c1_out), lambda b: (0, 0)),
            pl.BlockSpec((27 * c1_out, c2_out), lambda b: (0, 0)),
            pl.BlockSpec((c2_out, 1), lambda b: (0, 0)),
            pl.BlockSpec((c2_out, 1), lambda b: (0, 0)),
            pl.BlockSpec((1, Sp), lambda b: (0, 0)),
            pl.BlockSpec((Dpp, Dq * Dq), lambda b: (0, 0)),
        ],
        out_specs=pl.BlockSpec((1, c2_out, total_q), lambda b: (b, 0, 0)),
        scratch_shapes=[
            pltpu.VMEM((c1_out, Sp + 2 * P), jnp.bfloat16),
            pltpu.VMEM((c2_out, Sp + 2 * P), jnp.bfloat16),
            pltpu.VMEM((27, chunk), jnp.bfloat16),
            pltpu.VMEM((27 * c1_out, chunk), jnp.bfloat16),
            pltpu.VMEM((27 * c1_out, chunk), jnp.bfloat16),
        ],
        compiler_params=pltpu.CompilerParams(
            dimension_semantics=("parallel",),
            vmem_limit_bytes=_VMEM_LIMIT),
    )(x_flat, w1, w2, s2t, b2t, mask, _pool_sel(Dp))


def _make_pair_body(sp, p, dpp, offsets, chunk, nb, cin_a, cin_b):
    dp = round(dpp ** 0.5)

    def _body(x_ref, wa_ref, sa_ref, ba_ref, wb_ref, sb_ref, bb_ref,
              mask_ref, sel_ref, o_ref, mid_ref, act_ref, im_a_ref, im_b_ref):
        xs = [x_ref.at[e] for e in range(nb)]
        mids = [mid_ref.at[e] for e in range(nb)]
        acts = [act_ref.at[e] for e in range(nb)]
        _tconv(xs, wa_ref, sa_ref, ba_ref, mask_ref, mids, [im_a_ref],
               sp=sp, p=p, dpp=dpp, offsets=offsets, chunk=chunk,
               cin=cin_a, relu=True)
        _tconv(mids, wb_ref, sb_ref, bb_ref, mask_ref, acts, [im_b_ref],
               sp=sp, p=p, dpp=dpp, offsets=offsets, chunk=chunk,
               cin=cin_b, relu=True, masked=False)
        for e in range(nb):
            _pool_write(act_ref.at[e], o_ref.at[e], sel_ref, sp=sp, p=p, dp=dp)
    return _body


def _conv_pair_t(x_t, pa, pb, D, nb):
    B, cin, total = x_t.shape
    Dp = D + 2
    Sp = Dp ** 3
    P = Dp * Dp + Dp + 1
    Dpp = Dp * Dp
    assert total == Sp + 2 * P and B % nb == 0
    wat, sat, bat = pa
    wbt, sbt, bbt = pb
    ca_out = wat.shape[-1]
    cb_out = wbt.shape[-1]
    chunk = Sp - 2 * Dpp
    offsets = _tap_offsets(Dp, P)
    mask = _halo_mask_t(Dp)
    Do = D // 2
    if Do == 1:
        total_q = 1
    else:
        Dq = Do + 2
        total_q = Dq ** 3 + 2 * (Dq * Dq + Dq + 1)
    return pl.pallas_call(
        _make_pair_body(Sp, P, Dpp, offsets, chunk, nb, cin, ca_out),
        out_shape=jax.ShapeDtypeStruct((B, cb_out, total_q), jnp.bfloat16),
        grid=(B // nb,),
        in_specs=[
            pl.BlockSpec((nb, cin, total), lambda b: (b, 0, 0)),
            pl.BlockSpec((27 * cin, ca_out), lambda b: (0, 0)),
            pl.BlockSpec((ca_out, 1), lambda b: (0, 0)),
            pl.BlockSpec((ca_out, 1), lambda b: (0, 0)),
            pl.BlockSpec((27 * ca_out, cb_out), lambda b: (0, 0)),
            pl.BlockSpec((cb_out, 1), lambda b: (0, 0)),
            pl.BlockSpec((cb_out, 1), lambda b: (0, 0)),
            pl.BlockSpec((1, Sp), lambda b: (0, 0)),
            pl.BlockSpec((Dpp, (Do + 2) ** 2), lambda b: (0, 0)),
        ],
        out_specs=pl.BlockSpec((nb, cb_out, total_q), lambda b: (b, 0, 0)),
        scratch_shapes=[
            pltpu.VMEM((nb, ca_out, total), jnp.bfloat16),
            pltpu.VMEM((nb, cb_out, total), jnp.bfloat16),
            pltpu.VMEM((27 * cin, nb * chunk), jnp.bfloat16),
            pltpu.VMEM((27 * ca_out, nb * chunk), jnp.bfloat16),
        ],
        compiler_params=pltpu.CompilerParams(
            dimension_semantics=("parallel",),
            vmem_limit_bytes=_VMEM_LIMIT),
    )(x_t, wat, sat, bat, wbt, sbt, bbt, mask, _pool_sel(Dp))


def _head_body(x_ref, w9_ref, s9_ref, b9_ref, w10_ref, s10_ref, b10_ref,
               w11_ref, s11_ref, b11_ref, o_ref):
    h = jnp.dot(x_ref[...], w9_ref[...], preferred_element_type=jnp.float32)
    h = jnp.maximum(h * s9_ref[...] + b9_ref[...], 0.0)
    h = jnp.dot(h.astype(jnp.bfloat16), w10_ref[...],
                preferred_element_type=jnp.float32)
    h = jnp.maximum(h * s10_ref[...] + b10_ref[...], 0.0)
    h = jnp.dot(h.astype(jnp.bfloat16), w11_ref[...],
                preferred_element_type=jnp.float32)
    h = jnp.maximum(h * s11_ref[...] + b11_ref[...], 0.0)
    z = h - jnp.max(h, axis=1, keepdims=True)
    e = jnp.exp(z)
    o_ref[...] = e / jnp.sum(e, axis=1, keepdims=True)


def _head(v, head_params):
    B, C = v.shape
    (w9, s9, b9), (w10, s10, b10), (w11, s11, b11) = head_params
    nc = w11.shape[-1]
    return pl.pallas_call(
        _head_body,
        out_shape=jax.ShapeDtypeStruct((B, nc), jnp.float32),
        in_specs=[pl.BlockSpec(v.shape, lambda: (0, 0))] +
                 [pl.BlockSpec(a.shape, lambda: (0, 0))
                  for a in (w9, s9, b9, w10, s10, b10, w11, s11, b11)],
        out_specs=pl.BlockSpec((B, nc), lambda: (0, 0)),
        compiler_params=pltpu.CompilerParams(
            vmem_limit_bytes=_VMEM_LIMIT),
    )(v, w9, s9, b9, w10, s10, b10, w11, s11, b11)


@jax.jit
def _forward(x, params, head_params):
    B, D = x.shape[0], x.shape[1]
    xb = x.astype(jnp.bfloat16)
    Dp = D + 2
    Sp = Dp ** 3
    P = Dp * Dp + Dp + 1
    xp = jnp.pad(xb, ((0, 0), (1, 1), (1, 1), (1, 1)))
    x_flat = jnp.pad(xp.reshape(B, 1, Sp), ((0, 0), (0, 0), (P, P)))

    def tp(prm):
        w, s, b = prm
        return w, s.T, b.T

    h = _conv12_t(x_flat, params[0][0], params[1][0],
                  params[1][1].T, params[1][2].T, D)
    h = _conv_pair_t(h, tp(params[2]), tp(params[3]), D // 2,
                     nb=math.gcd(2, B))
    h = _conv_pair_t(h, tp(params[4]), tp(params[5]), D // 4,
                     nb=math.gcd(8, B))
    h = _conv_pair_t(h, tp(params[6]), tp(params[7]), D // 8,
                     nb=math.gcd(16, B))
    return _head(h.reshape(B, 256), head_params)


def kernel(x, w0, s0, sh0, w1, s1, sh1, w2, s2, sh2, w3, s3, sh3,
           w4, s4, sh4, w5, s5, sh5, w6, s6, sh6, w7, s7, sh7,
           w8, s8, sh8, w9, s9, sh9, w10, s10, sh10,
           hw0, hs0, hb0, hw1, hs1, hb1, hw2, hs2, hb2):
    params = [(w0, s0, sh0), (w1, s1, sh1), (w2, s2, sh2), (w3, s3, sh3),
              (w4, s4, sh4), (w5, s5, sh5), (w6, s6, sh6), (w7, s7, sh7)]
    head_params = ((hw0, hs0, hb0), (hw1, hs1, hb1), (hw2, hs2, hb2))
    return _forward(x, params, head_params)
